# baseline (speedup 1.0000x reference)
import jax
import jax.numpy as jnp
from jax.experimental import pallas as pl

_B, _N, _DIN, _DOUT, _H = 8, 512, 64, 64, 8
_C = _DOUT // _H
_NEG_SLOPE = 0.2


_LOG2E = 1.4426950408889634


_GPP = 4


def _gat_graph_kernel(adj_ref, x_ref, w_ref, asrc_ref, adst_ref, bias_ref,
                      out_ref):
    asrc = asrc_ref[...] * _LOG2E
    adst = adst_ref[...] * _LOG2E

    seg = (jax.lax.broadcasted_iota(jnp.int32, (_H * _C, _H), 0) // _C
           == jax.lax.broadcasted_iota(jnp.int32, (_H * _C, _H), 1)
           ).astype(jnp.float32)
    colsel = (jax.lax.broadcasted_iota(jnp.int32, (_H * _C, _C), 0) % _C
              == jax.lax.broadcasted_iota(jnp.int32, (_H * _C, _C), 1)
              ).astype(jnp.float32)
    ones_c1 = jnp.ones((_C, 1), dtype=jnp.float32)

    def _blockdiag(att):
        tmp = jnp.dot(seg, att, preferred_element_type=jnp.float32)
        flat = jnp.dot(tmp * colsel, ones_c1,
                       preferred_element_type=jnp.float32)
        return seg * flat

    bd_src = _blockdiag(asrc)
    bd_dst = _blockdiag(adst)
    row_s = jax.lax.broadcasted_iota(jnp.int32, (_N, _N), 0)
    col_t = jax.lax.broadcasted_iota(jnp.int32, (_N, _N), 1)
    eye = row_s == col_t
    ones_c = jnp.ones((_N, _C), dtype=jnp.bfloat16)

    for i in range(_GPP):
        xt = x_ref[i]
        xw = jax.lax.dot_general(
            xt, w_ref[...],
            dimension_numbers=(((0,), (0,)), ((), ())),
            preferred_element_type=jnp.float32)
        a_src = jnp.dot(xw, bd_src,
                        preferred_element_type=jnp.float32)
        a_dstT = jax.lax.dot_general(
            bd_dst, xw,
            dimension_numbers=(((0,), (1,)), ((), ())),
            preferred_element_type=jnp.float32)

        u1 = a_src.astype(jnp.bfloat16)
        v1 = a_dstT.astype(jnp.bfloat16)
        xwb = xw.astype(jnp.bfloat16)

        mask = (adj_ref[i] != 0) | eye

        mms = []
        for h in range(_H):
            l = u1[:, h:h + 1] + v1[h:h + 1, :]
            e = jnp.exp2(jnp.maximum(l, jnp.bfloat16(_NEG_SLOPE) * l))
            e = jnp.where(mask, e, jnp.bfloat16(0.0))
            g = jnp.concatenate([xwb[:, h * _C:(h + 1) * _C], ones_c],
                                axis=1)
            mms.append(jax.lax.dot_general(
                g, e,
                dimension_numbers=(((0,), (0,)), ((), ())),
                preferred_element_type=jnp.float32))

        num = jnp.concatenate([m[:_C] for m in mms], axis=0)
        den = jnp.concatenate([m[_C:] for m in mms], axis=0)
        y = num / den + bias_ref[...][:, None]
        y = jnp.where(y > 0, y,
                      jnp.exp2(jnp.minimum(y, 0.0) * _LOG2E) - 1.0)
        out_ref[i] = y


def kernel(features_batch, adj_mats_batch, W, att_src, att_dst, bias):
    xt = features_batch.transpose(0, 2, 1)

    out = pl.pallas_call(
        _gat_graph_kernel,
        grid=(_B // _GPP,),
        in_specs=[
            pl.BlockSpec((_GPP, _N, _N), lambda b: (b, 0, 0)),
            pl.BlockSpec((_GPP, _DIN, _N), lambda b: (b, 0, 0)),
            pl.BlockSpec((_DIN, _H * _C), lambda b: (0, 0)),
            pl.BlockSpec((_H, _C), lambda b: (0, 0)),
            pl.BlockSpec((_H, _C), lambda b: (0, 0)),
            pl.BlockSpec((_DOUT,), lambda b: (0,)),
        ],
        out_specs=pl.BlockSpec((_GPP, _DOUT, _N), lambda b: (b, 0, 0)),
        out_shape=jax.ShapeDtypeStruct((_B, _DOUT, _N), jnp.float32),
    )(adj_mats_batch, xt, W, att_src, att_dst, bias)
    return out.transpose(0, 2, 1)
